# Initial kernel scaffold; baseline (speedup 1.0000x reference)
import jax
import jax.numpy as jnp
from jax.experimental import pallas as pl


def kernel(nodes, edges, lin1_w, lin1_b, bn1_g, bn1_b, bn1_m, bn1_v, gcn1_w, gcn1_b, lin2_w, lin2_b, bn2_g, bn2_b, bn2_m, bn2_v, gcn2_w, gcn2_b):
    raise NotImplementedError("write your pallas kernel here")


# speedup vs baseline: 9.0394x; 9.0394x over previous
import functools

import jax
import jax.numpy as jnp
from jax import lax
from jax.experimental import pallas as pl
from jax.experimental.pallas import tpu as pltpu
from jax.experimental.pallas import tpu_sc as plsc

N = 10000
D = 128
H = 128
C = 40
CP = 128
EPS = 1e-5

NC, NS, LANES = 2, 16, 16
NPAD = 10240
RPT = NPAD // NS
E_RAW = 320000
CHUNK = 128
EPT_CHUNKS = 80
EPT = CHUNK * EPT_CHUNKS
EPAD = EPT * NC * NS
ROWS_TOTAL = EPAD // CHUNK
ROWS_PER_SC = ROWS_TOTAL // NC

_MESH = plsc.VectorSubcoreMesh(
    core_axis_name="c", subcore_axis_name="s", num_cores=NC, num_subcores=NS)


def _deg_body(dst_hbm, ones_hbm, zeros_hbm, out_hbm, dst_v, ones_v, acc, sem):
    c = lax.axis_index("c")
    s = lax.axis_index("s")
    row0 = c * ROWS_PER_SC + s * EPT_CHUNKS
    pltpu.sync_copy(dst_hbm.at[pl.ds(row0, EPT_CHUNKS)], dst_v)
    pltpu.sync_copy(ones_hbm, ones_v)
    pltpu.sync_copy(zeros_hbm, acc.at[pl.ds(s * RPT, RPT)])
    plsc.subcore_barrier()

    def body(j, carry):
        pltpu.sync_copy(ones_v, acc.at[dst_v.at[j]], add=True)
        return carry

    lax.fori_loop(0, EPT_CHUNKS, body, 0)
    plsc.subcore_barrier()
    pltpu.sync_copy(acc.at[pl.ds(s * RPT, RPT)],
                    out_hbm.at[c, pl.ds(s * RPT, RPT)])


_deg_call = pl.kernel(
    _deg_body,
    out_type=jax.ShapeDtypeStruct((NC, NPAD, H), jnp.float32),
    mesh=_MESH,
    scratch_types=[
        pltpu.VMEM((EPT_CHUNKS, CHUNK), jnp.int32),
        pltpu.VMEM((CHUNK, H), jnp.float32),
        pltpu.VMEM_SHARED((NPAD, H), jnp.float32),
        pltpu.SemaphoreType.DMA,
    ],
)


def _conv_body(y_hbm, src_hbm, dst_hbm, zeros_hbm, out_hbm,
               src_v, dst_v, rows_v, acc, sem):
    c = lax.axis_index("c")
    s = lax.axis_index("s")
    row0 = c * ROWS_PER_SC + s * EPT_CHUNKS
    pltpu.sync_copy(src_hbm.at[pl.ds(row0, EPT_CHUNKS)], src_v)
    pltpu.sync_copy(dst_hbm.at[pl.ds(row0, EPT_CHUNKS)], dst_v)
    pltpu.sync_copy(zeros_hbm, acc.at[pl.ds(s * RPT, RPT)])
    plsc.subcore_barrier()

    def body(j, carry):
        pltpu.async_copy(y_hbm.at[src_v.at[j]], rows_v, sem).wait()
        pltpu.sync_copy(rows_v, acc.at[dst_v.at[j]], add=True)
        return carry

    lax.fori_loop(0, EPT_CHUNKS, body, 0)
    plsc.subcore_barrier()
    pltpu.sync_copy(acc.at[pl.ds(s * RPT, RPT)],
                    out_hbm.at[c, pl.ds(s * RPT, RPT)])


def _make_conv(width):
    return pl.kernel(
        _conv_body,
        out_type=jax.ShapeDtypeStruct((NC, NPAD, width), jnp.float32),
        mesh=_MESH,
        scratch_types=[
            pltpu.VMEM((EPT_CHUNKS, CHUNK), jnp.int32),
            pltpu.VMEM((EPT_CHUNKS, CHUNK), jnp.int32),
            pltpu.VMEM((CHUNK, width), jnp.float32),
            pltpu.VMEM_SHARED((NPAD, width), jnp.float32),
            pltpu.SemaphoreType.DMA,
        ],
    )


_conv128 = _make_conv(H)

_TC_R = 1000


def _dinv_of(degp):
    deg = degp[0] + degp[1] + 1.0
    return lax.rsqrt(deg[:, 0:1])


def _leaky(x):
    return jnp.where(x > 0, x, 0.01 * x)


def _dense1_body(nodes_ref, w1_ref, b1_ref, gw_ref, degp_ref, y_ref):
    x = jnp.dot(nodes_ref[...], w1_ref[...],
                preferred_element_type=jnp.float32) + b1_ref[...]
    x = _leaky(x)
    dinv = _dinv_of(degp_ref[...])
    y_ref[...] = jnp.dot(x, gw_ref[...],
                         preferred_element_type=jnp.float32) * dinv


def _dense2_body(p_ref, y1_ref, degp_ref, w2_ref, b2_ref, g1b_ref, gw2_ref,
                 y2_ref):
    dinv = _dinv_of(degp_ref[...])
    h = (p_ref[0] + p_ref[1] + y1_ref[...]) * dinv + g1b_ref[...]
    x2 = _leaky(jnp.dot(h, w2_ref[...],
                        preferred_element_type=jnp.float32) + b2_ref[...])
    y2_ref[...] = jnp.dot(x2, gw2_ref[...],
                          preferred_element_type=jnp.float32) * dinv


def _final_body(p_ref, y2_ref, degp_ref, g2b_ref, out_ref):
    dinv = _dinv_of(degp_ref[...])
    o = (p_ref[0] + p_ref[1] + y2_ref[...]) * dinv + g2b_ref[...]
    col = lax.broadcasted_iota(jnp.int32, (_TC_R, CP), 1)
    valid = col < C
    m = jnp.max(jnp.where(valid, o, -1e30), axis=1, keepdims=True)
    e = jnp.where(valid, jnp.exp(o - m), 0.0)
    lse = jnp.log(jnp.sum(e, axis=1, keepdims=True))
    out_ref[...] = (o - m - lse)[:, :C]


def _row_spec(width):
    return pl.BlockSpec((_TC_R, width), lambda i: (i, 0))


def _pair_spec(width):
    return pl.BlockSpec((2, _TC_R, width), lambda i: (0, i, 0))


def _full_spec(shape):
    return pl.BlockSpec(shape, lambda i: tuple(0 for _ in shape))


_GRID = (N // _TC_R,)

_dense1_call = pl.pallas_call(
    _dense1_body,
    grid=_GRID,
    in_specs=[_row_spec(D), _full_spec((D, D)), _full_spec((1, D)),
              _full_spec((D, H)), _pair_spec(LANES)],
    out_specs=_row_spec(H),
    out_shape=jax.ShapeDtypeStruct((N, H), jnp.float32),
)

_dense2_call = pl.pallas_call(
    _dense2_body,
    grid=_GRID,
    in_specs=[_pair_spec(H), _row_spec(H), _pair_spec(LANES),
              _full_spec((H, H)), _full_spec((1, H)), _full_spec((1, H)),
              _full_spec((H, CP))],
    out_specs=_row_spec(CP),
    out_shape=jax.ShapeDtypeStruct((N, CP), jnp.float32),
)

_final_call = pl.pallas_call(
    _final_body,
    grid=_GRID,
    in_specs=[_pair_spec(CP), _row_spec(CP), _pair_spec(LANES),
              _full_spec((1, CP))],
    out_specs=_row_spec(C),
    out_shape=jax.ShapeDtypeStruct((N, C), jnp.float32),
)


def kernel(nodes, edges, lin1_w, lin1_b, bn1_g, bn1_b, bn1_m, bn1_v,
           gcn1_w, gcn1_b, lin2_w, lin2_b, bn2_g, bn2_b, bn2_m, bn2_v,
           gcn2_w, gcn2_b):
    s1 = bn1_g * lax.rsqrt(bn1_v + EPS)
    w1f = lin1_w * s1[None, :]
    b1f = (lin1_b * s1 + (bn1_b - bn1_m * s1))[None, :]
    s2 = bn2_g * lax.rsqrt(bn2_v + EPS)
    w2f = lin2_w * s2[None, :]
    b2f = (lin2_b * s2 + (bn2_b - bn2_m * s2))[None, :]
    gw2p = jnp.pad(gcn2_w, ((0, 0), (0, CP - C)))
    g2bp = jnp.pad(gcn2_b, (0, CP - C))[None, :]

    pad = EPAD - E_RAW
    src2d = jnp.concatenate(
        [edges[:, 0], jnp.zeros((pad,), jnp.int32)]).reshape(ROWS_TOTAL, CHUNK)
    dst2d = jnp.concatenate(
        [edges[:, 1], jnp.full((pad,), N, jnp.int32)]).reshape(ROWS_TOTAL, CHUNK)

    ones128 = jnp.ones((CHUNK, H), jnp.float32)
    z128 = jnp.zeros((RPT, H), jnp.float32)

    degp = _deg_call(dst2d, ones128, z128)
    degp_n = degp[:, :N, :LANES]
    y1 = _dense1_call(nodes, w1f, b1f, gcn1_w, degp_n)
    p1 = _conv128(y1, src2d, dst2d, z128)
    y2 = _dense2_call(p1[:, :N], y1, degp_n, w2f, b2f,
                      gcn1_b[None, :], gw2p)
    p2 = _conv128(y2, src2d, dst2d, z128)
    return _final_call(p2[:, :N], y2, degp_n, g2bp)


# speedup vs baseline: 9.7935x; 1.0834x over previous
import functools

import jax
import jax.numpy as jnp
from jax import lax
from jax.experimental import pallas as pl
from jax.experimental.pallas import tpu as pltpu
from jax.experimental.pallas import tpu_sc as plsc

N = 10000
D = 128
H = 128
C = 40
CP = 128
EPS = 1e-5

NC, NS, LANES = 2, 16, 16
NPAD = 10240
RPT = NPAD // NS
E_RAW = 320000
CHUNK = 128
EPT_CHUNKS = 80
EPT = CHUNK * EPT_CHUNKS
EPAD = EPT * NC * NS
ROWS_TOTAL = EPAD // CHUNK
ROWS_PER_SC = ROWS_TOTAL // NC

_MESH = plsc.VectorSubcoreMesh(
    core_axis_name="c", subcore_axis_name="s", num_cores=NC, num_subcores=NS)


def _deg_body(dst_hbm, ones_hbm, zeros_hbm, out_hbm, dst_v, ones_v, acc, sem):
    c = lax.axis_index("c")
    s = lax.axis_index("s")
    row0 = c * ROWS_PER_SC + s * EPT_CHUNKS
    pltpu.sync_copy(dst_hbm.at[pl.ds(row0, EPT_CHUNKS)], dst_v)
    pltpu.sync_copy(ones_hbm, ones_v)
    pltpu.sync_copy(zeros_hbm, acc.at[pl.ds(s * RPT, RPT)])
    plsc.subcore_barrier()

    def body(j, carry):
        pltpu.sync_copy(ones_v, acc.at[dst_v.at[j]], add=True)
        return carry

    lax.fori_loop(0, EPT_CHUNKS, body, 0)
    plsc.subcore_barrier()
    pltpu.sync_copy(acc.at[pl.ds(s * RPT, RPT)],
                    out_hbm.at[c, pl.ds(s * RPT, RPT)])


_deg_call = pl.kernel(
    _deg_body,
    out_type=jax.ShapeDtypeStruct((NC, NPAD, H), jnp.float32),
    mesh=_MESH,
    scratch_types=[
        pltpu.VMEM((EPT_CHUNKS, CHUNK), jnp.int32),
        pltpu.VMEM((CHUNK, H), jnp.float32),
        pltpu.VMEM_SHARED((NPAD, H), jnp.float32),
        pltpu.SemaphoreType.DMA,
    ],
)


_HALF = EPT_CHUNKS // 2


def _conv_body(y_hbm, src_hbm, dst_hbm, zeros_hbm, out_hbm,
               src_v, dst_v, rows_v, acc, sem_g):
    c = lax.axis_index("c")
    s = lax.axis_index("s")
    row0 = c * ROWS_PER_SC + s * EPT_CHUNKS
    pltpu.sync_copy(zeros_hbm, acc.at[pl.ds(s * RPT, RPT)])
    plsc.subcore_barrier()

    def _gather(j, slot):
        return pltpu.make_async_copy(y_hbm.at[src_v.at[j]],
                                     rows_v.at[slot], sem_g)

    for h in range(2):
        pltpu.sync_copy(src_hbm.at[pl.ds(row0 + h * _HALF, _HALF)], src_v)
        pltpu.sync_copy(dst_hbm.at[pl.ds(row0 + h * _HALF, _HALF)], dst_v)
        _gather(0, 0).start()

        def pair(k, carry):
            j0 = 2 * k
            _gather(j0, 0).wait()
            _gather(j0 + 1, 1).start()
            pltpu.sync_copy(rows_v.at[0], acc.at[dst_v.at[j0]], add=True)
            _gather(j0 + 1, 1).wait()

            @pl.when(k < _HALF // 2 - 1)
            def _():
                _gather(j0 + 2, 0).start()

            pltpu.sync_copy(rows_v.at[1], acc.at[dst_v.at[j0 + 1]], add=True)
            return carry

        lax.fori_loop(0, _HALF // 2, pair, 0)

    plsc.subcore_barrier()
    pltpu.sync_copy(acc.at[pl.ds(s * RPT, RPT)],
                    out_hbm.at[c, pl.ds(s * RPT, RPT)])


def _make_conv(width):
    return pl.kernel(
        _conv_body,
        out_type=jax.ShapeDtypeStruct((NC, NPAD, width), jnp.float32),
        mesh=_MESH,
        scratch_types=[
            pltpu.VMEM((_HALF, CHUNK), jnp.int32),
            pltpu.VMEM((_HALF, CHUNK), jnp.int32),
            pltpu.VMEM((2, CHUNK, width), jnp.float32),
            pltpu.VMEM_SHARED((NPAD, width), jnp.float32),
            pltpu.SemaphoreType.DMA,
        ],
    )


_conv128 = _make_conv(H)

_TC_R = 1000


def _dinv_of(degp):
    deg = degp[0] + degp[1] + 1.0
    return lax.rsqrt(deg[:, 0:1])


def _leaky(x):
    return jnp.where(x > 0, x, 0.01 * x)


def _dense1_body(nodes_ref, w1_ref, b1_ref, gw_ref, degp_ref, y_ref):
    x = jnp.dot(nodes_ref[...], w1_ref[...],
                preferred_element_type=jnp.float32) + b1_ref[...]
    x = _leaky(x)
    dinv = _dinv_of(degp_ref[...])
    y_ref[...] = jnp.dot(x, gw_ref[...],
                         preferred_element_type=jnp.float32) * dinv


def _dense2_body(p_ref, y1_ref, degp_ref, w2_ref, b2_ref, g1b_ref, gw2_ref,
                 y2_ref):
    dinv = _dinv_of(degp_ref[...])
    h = (p_ref[0] + p_ref[1] + y1_ref[...]) * dinv + g1b_ref[...]
    x2 = _leaky(jnp.dot(h, w2_ref[...],
                        preferred_element_type=jnp.float32) + b2_ref[...])
    y2_ref[...] = jnp.dot(x2, gw2_ref[...],
                          preferred_element_type=jnp.float32) * dinv


def _final_body(p_ref, y2_ref, degp_ref, g2b_ref, out_ref):
    dinv = _dinv_of(degp_ref[...])
    o = (p_ref[0] + p_ref[1] + y2_ref[...]) * dinv + g2b_ref[...]
    col = lax.broadcasted_iota(jnp.int32, (_TC_R, CP), 1)
    valid = col < C
    m = jnp.max(jnp.where(valid, o, -1e30), axis=1, keepdims=True)
    e = jnp.where(valid, jnp.exp(o - m), 0.0)
    lse = jnp.log(jnp.sum(e, axis=1, keepdims=True))
    out_ref[...] = (o - m - lse)[:, :C]


def _row_spec(width):
    return pl.BlockSpec((_TC_R, width), lambda i: (i, 0))


def _pair_spec(width):
    return pl.BlockSpec((2, _TC_R, width), lambda i: (0, i, 0))


def _full_spec(shape):
    return pl.BlockSpec(shape, lambda i: tuple(0 for _ in shape))


_GRID = (N // _TC_R,)

_dense1_call = pl.pallas_call(
    _dense1_body,
    grid=_GRID,
    in_specs=[_row_spec(D), _full_spec((D, D)), _full_spec((1, D)),
              _full_spec((D, H)), _pair_spec(LANES)],
    out_specs=_row_spec(H),
    out_shape=jax.ShapeDtypeStruct((N, H), jnp.float32),
)

_dense2_call = pl.pallas_call(
    _dense2_body,
    grid=_GRID,
    in_specs=[_pair_spec(H), _row_spec(H), _pair_spec(LANES),
              _full_spec((H, H)), _full_spec((1, H)), _full_spec((1, H)),
              _full_spec((H, CP))],
    out_specs=_row_spec(CP),
    out_shape=jax.ShapeDtypeStruct((N, CP), jnp.float32),
)

_final_call = pl.pallas_call(
    _final_body,
    grid=_GRID,
    in_specs=[_pair_spec(CP), _row_spec(CP), _pair_spec(LANES),
              _full_spec((1, CP))],
    out_specs=_row_spec(C),
    out_shape=jax.ShapeDtypeStruct((N, C), jnp.float32),
)


def kernel(nodes, edges, lin1_w, lin1_b, bn1_g, bn1_b, bn1_m, bn1_v,
           gcn1_w, gcn1_b, lin2_w, lin2_b, bn2_g, bn2_b, bn2_m, bn2_v,
           gcn2_w, gcn2_b):
    s1 = bn1_g * lax.rsqrt(bn1_v + EPS)
    w1f = lin1_w * s1[None, :]
    b1f = (lin1_b * s1 + (bn1_b - bn1_m * s1))[None, :]
    s2 = bn2_g * lax.rsqrt(bn2_v + EPS)
    w2f = lin2_w * s2[None, :]
    b2f = (lin2_b * s2 + (bn2_b - bn2_m * s2))[None, :]
    gw2p = jnp.pad(gcn2_w, ((0, 0), (0, CP - C)))
    g2bp = jnp.pad(gcn2_b, (0, CP - C))[None, :]

    pad = EPAD - E_RAW
    src2d = jnp.concatenate(
        [edges[:, 0], jnp.zeros((pad,), jnp.int32)]).reshape(ROWS_TOTAL, CHUNK)
    dst2d = jnp.concatenate(
        [edges[:, 1], jnp.full((pad,), N, jnp.int32)]).reshape(ROWS_TOTAL, CHUNK)

    ones128 = jnp.ones((CHUNK, H), jnp.float32)
    z128 = jnp.zeros((RPT, H), jnp.float32)

    degp = _deg_call(dst2d, ones128, z128)
    degp_n = degp[:, :N, :LANES]
    y1 = _dense1_call(nodes, w1f, b1f, gcn1_w, degp_n)
    p1 = _conv128(y1, src2d, dst2d, z128)
    y2 = _dense2_call(p1[:, :N], y1, degp_n, w2f, b2f,
                      gcn1_b[None, :], gw2p)
    p2 = _conv128(y2, src2d, dst2d, z128)
    return _final_call(p2[:, :N], y2, degp_n, g2bp)


# speedup vs baseline: 22.4354x; 2.2908x over previous
import functools

import jax
import jax.numpy as jnp
from jax import lax
from jax.experimental import pallas as pl
from jax.experimental.pallas import tpu as pltpu
from jax.experimental.pallas import tpu_sc as plsc

N = 10000
D = 128
H = 128
C = 40
CP = 128
EPS = 1e-5

NC, NS, LANES = 2, 16, 16
NPAD = 10240
RPT = NPAD // NS
E_RAW = 320000
CHUNK = 128
EPT_CHUNKS = 80
EPT = CHUNK * EPT_CHUNKS
EPAD = EPT * NC * NS
ROWS_TOTAL = EPAD // CHUNK
ROWS_PER_SC = ROWS_TOTAL // NC

_MESH = plsc.VectorSubcoreMesh(
    core_axis_name="c", subcore_axis_name="s", num_cores=NC, num_subcores=NS)


def _deg_body(dst_hbm, ones_hbm, zeros_hbm, out_hbm, dst_v, ones_v, acc, sem):
    c = lax.axis_index("c")
    s = lax.axis_index("s")
    row0 = c * ROWS_PER_SC + s * EPT_CHUNKS
    pltpu.sync_copy(dst_hbm.at[pl.ds(row0, EPT_CHUNKS)], dst_v)
    pltpu.sync_copy(ones_hbm, ones_v)
    pltpu.sync_copy(zeros_hbm, acc.at[pl.ds(s * RPT, RPT)])
    plsc.subcore_barrier()

    def body(j, carry):
        pltpu.sync_copy(ones_v, acc.at[dst_v.at[j]], add=True)
        return carry

    lax.fori_loop(0, EPT_CHUNKS, body, 0)
    plsc.subcore_barrier()
    pltpu.sync_copy(acc.at[pl.ds(s * RPT, RPT)],
                    out_hbm.at[c, pl.ds(s * RPT, RPT)])


_deg_call = pl.kernel(
    _deg_body,
    out_type=jax.ShapeDtypeStruct((NC, NPAD, H), jnp.float32),
    mesh=_MESH,
    scratch_types=[
        pltpu.VMEM((EPT_CHUNKS, CHUNK), jnp.int32),
        pltpu.VMEM((CHUNK, H), jnp.float32),
        pltpu.VMEM_SHARED((NPAD, H), jnp.float32),
        pltpu.SemaphoreType.DMA,
    ],
)


_HALF = EPT_CHUNKS // 2


def _conv_body(y_hbm, src_hbm, dst_hbm, zeros_hbm, out_hbm,
               src_v, dst_v, rows_v, acc, sem_g):
    c = lax.axis_index("c")
    s = lax.axis_index("s")
    row0 = c * ROWS_PER_SC + s * EPT_CHUNKS
    pltpu.sync_copy(zeros_hbm, acc.at[pl.ds(s * RPT, RPT)])
    plsc.subcore_barrier()

    def _gather(j, slot):
        return pltpu.make_async_copy(y_hbm.at[src_v.at[j]],
                                     rows_v.at[slot], sem_g)

    for h in range(2):
        pltpu.sync_copy(src_hbm.at[pl.ds(row0 + h * _HALF, _HALF)], src_v)
        pltpu.sync_copy(dst_hbm.at[pl.ds(row0 + h * _HALF, _HALF)], dst_v)
        _gather(0, 0).start()

        def pair(k, carry):
            j0 = 2 * k
            _gather(j0, 0).wait()
            _gather(j0 + 1, 1).start()
            pltpu.sync_copy(rows_v.at[0], acc.at[dst_v.at[j0]], add=True)
            _gather(j0 + 1, 1).wait()

            @pl.when(k < _HALF // 2 - 1)
            def _():
                _gather(j0 + 2, 0).start()

            pltpu.sync_copy(rows_v.at[1], acc.at[dst_v.at[j0 + 1]], add=True)
            return carry

        lax.fori_loop(0, _HALF // 2, pair, 0)

    plsc.subcore_barrier()
    pltpu.sync_copy(acc.at[pl.ds(s * RPT, RPT)],
                    out_hbm.at[c, pl.ds(s * RPT, RPT)])


def _make_conv(width):
    return pl.kernel(
        _conv_body,
        out_type=jax.ShapeDtypeStruct((NC, NPAD, width), jnp.float32),
        mesh=_MESH,
        scratch_types=[
            pltpu.VMEM((_HALF, CHUNK), jnp.int32),
            pltpu.VMEM((_HALF, CHUNK), jnp.int32),
            pltpu.VMEM((2, CHUNK, width), jnp.float32),
            pltpu.VMEM_SHARED((NPAD, width), jnp.float32),
            pltpu.SemaphoreType.DMA,
        ],
    )


_conv128 = _make_conv(H)

_TC_R = 1000


def _dinv_of(degp):
    deg = degp[0] + degp[1] + 1.0
    return lax.rsqrt(deg[:, 0:1])


def _leaky(x):
    return jnp.where(x > 0, x, 0.01 * x)


def _dense1_body(nodes_ref, w1_ref, b1_ref, gw_ref, degp_ref, y_ref):
    x = jnp.dot(nodes_ref[...], w1_ref[...],
                preferred_element_type=jnp.float32) + b1_ref[...]
    x = _leaky(x)
    dinv = _dinv_of(degp_ref[...])
    y_ref[...] = jnp.dot(x, gw_ref[...],
                         preferred_element_type=jnp.float32) * dinv


def _dense2_body(p_ref, y1_ref, degp_ref, w2_ref, b2_ref, g1b_ref, gw2_ref,
                 y2_ref):
    dinv = _dinv_of(degp_ref[...])
    h = (p_ref[0] + p_ref[1] + y1_ref[...]) * dinv + g1b_ref[...]
    x2 = _leaky(jnp.dot(h, w2_ref[...],
                        preferred_element_type=jnp.float32) + b2_ref[...])
    y2_ref[...] = jnp.dot(x2, gw2_ref[...],
                          preferred_element_type=jnp.float32) * dinv


def _final_body(p_ref, y2_ref, degp_ref, g2b_ref, out_ref):
    dinv = _dinv_of(degp_ref[...])
    o = (p_ref[0] + p_ref[1] + y2_ref[...]) * dinv + g2b_ref[...]
    col = lax.broadcasted_iota(jnp.int32, (_TC_R, CP), 1)
    valid = col < C
    m = jnp.max(jnp.where(valid, o, -1e30), axis=1, keepdims=True)
    e = jnp.where(valid, jnp.exp(o - m), 0.0)
    lse = jnp.log(jnp.sum(e, axis=1, keepdims=True))
    out_ref[...] = (o - m - lse)[:, :C]


def _row_spec(width):
    return pl.BlockSpec((_TC_R, width), lambda i: (i, 0))


def _pair_spec(width):
    return pl.BlockSpec((2, _TC_R, width), lambda i: (0, i, 0))


def _full_spec(shape):
    return pl.BlockSpec(shape, lambda i: tuple(0 for _ in shape))


_GRID = (N // _TC_R,)

_dense1_call = pl.pallas_call(
    _dense1_body,
    grid=_GRID,
    in_specs=[_row_spec(D), _full_spec((D, D)), _full_spec((1, D)),
              _full_spec((D, H)), _pair_spec(LANES)],
    out_specs=_row_spec(H),
    out_shape=jax.ShapeDtypeStruct((N, H), jnp.float32),
)

_dense2_call = pl.pallas_call(
    _dense2_body,
    grid=_GRID,
    in_specs=[_pair_spec(H), _row_spec(H), _pair_spec(LANES),
              _full_spec((H, H)), _full_spec((1, H)), _full_spec((1, H)),
              _full_spec((H, CP))],
    out_specs=_row_spec(CP),
    out_shape=jax.ShapeDtypeStruct((N, CP), jnp.float32),
)

_final_call = pl.pallas_call(
    _final_body,
    grid=_GRID,
    in_specs=[_pair_spec(CP), _row_spec(CP), _pair_spec(LANES),
              _full_spec((1, CP))],
    out_specs=_row_spec(C),
    out_shape=jax.ShapeDtypeStruct((N, C), jnp.float32),
)


def kernel(nodes, edges, lin1_w, lin1_b, bn1_g, bn1_b, bn1_m, bn1_v,
           gcn1_w, gcn1_b, lin2_w, lin2_b, bn2_g, bn2_b, bn2_m, bn2_v,
           gcn2_w, gcn2_b):
    s1 = bn1_g * lax.rsqrt(bn1_v + EPS)
    w1f = lin1_w * s1[None, :]
    b1f = (lin1_b * s1 + (bn1_b - bn1_m * s1))[None, :]
    s2 = bn2_g * lax.rsqrt(bn2_v + EPS)
    w2f = lin2_w * s2[None, :]
    b2f = (lin2_b * s2 + (bn2_b - bn2_m * s2))[None, :]
    gw2p = jnp.pad(gcn2_w, ((0, 0), (0, CP - C)))
    g2bp = jnp.pad(gcn2_b, (0, CP - C))[None, :]

    half_pad = (EPAD - E_RAW) // 2
    half_e = E_RAW // 2
    pad_src = (jnp.arange(half_pad, dtype=jnp.int32) * 7) % N
    pad_dst = N + (jnp.arange(half_pad, dtype=jnp.int32) % (NPAD - N))
    src_flat = jnp.concatenate(
        [edges[:half_e, 0], pad_src, edges[half_e:, 0], pad_src])
    dst_flat = jnp.concatenate(
        [edges[:half_e, 1], pad_dst, edges[half_e:, 1], pad_dst])
    src2d = src_flat.reshape(ROWS_TOTAL, CHUNK)
    dst2d = dst_flat.reshape(ROWS_TOTAL, CHUNK)

    ones128 = jnp.ones((CHUNK, H), jnp.float32)
    z128 = jnp.zeros((RPT, H), jnp.float32)

    degp = _deg_call(dst2d, ones128, z128)
    degp_n = degp[:, :N, :LANES]
    y1 = _dense1_call(nodes, w1f, b1f, gcn1_w, degp_n)
    p1 = _conv128(y1, src2d, dst2d, z128)
    y2 = _dense2_call(p1[:, :N], y1, degp_n, w2f, b2f,
                      gcn1_b[None, :], gw2p)
    p2 = _conv128(y2, src2d, dst2d, z128)
    return _final_call(p2[:, :N], y2, degp_n, g2bp)


# speedup vs baseline: 22.4793x; 1.0020x over previous
import functools

import jax
import jax.numpy as jnp
from jax import lax
from jax.experimental import pallas as pl
from jax.experimental.pallas import tpu as pltpu
from jax.experimental.pallas import tpu_sc as plsc

N = 10000
D = 128
H = 128
C = 40
CP = 128
EPS = 1e-5

NC, NS, LANES = 2, 16, 16
NPAD = 10240
RPT = NPAD // NS
E_RAW = 320000
CHUNK = 128
EPT_CHUNKS = 80
EPT = CHUNK * EPT_CHUNKS
EPAD = EPT * NC * NS
ROWS_TOTAL = EPAD // CHUNK
ROWS_PER_SC = ROWS_TOTAL // NC

_MESH = plsc.VectorSubcoreMesh(
    core_axis_name="c", subcore_axis_name="s", num_cores=NC, num_subcores=NS)


def _deg_body(dst_hbm, ones_hbm, zeros_hbm, out_hbm, dst_v, ones_v, acc, sem):
    c = lax.axis_index("c")
    s = lax.axis_index("s")
    row0 = c * ROWS_PER_SC + s * EPT_CHUNKS
    pltpu.sync_copy(dst_hbm.at[pl.ds(row0, EPT_CHUNKS)], dst_v)
    pltpu.sync_copy(ones_hbm, ones_v)
    pltpu.sync_copy(zeros_hbm, acc.at[pl.ds(s * RPT, RPT)])
    plsc.subcore_barrier()

    def body(j, carry):
        pltpu.sync_copy(ones_v, acc.at[dst_v.at[j]], add=True)
        return carry

    lax.fori_loop(0, EPT_CHUNKS, body, 0)
    plsc.subcore_barrier()
    pltpu.sync_copy(acc.at[pl.ds(s * RPT, RPT)],
                    out_hbm.at[c, pl.ds(s * RPT, RPT)])


_deg_call = pl.kernel(
    _deg_body,
    out_type=jax.ShapeDtypeStruct((NC, NPAD, H), jnp.float32),
    mesh=_MESH,
    scratch_types=[
        pltpu.VMEM((EPT_CHUNKS, CHUNK), jnp.int32),
        pltpu.VMEM((CHUNK, H), jnp.float32),
        pltpu.VMEM_SHARED((NPAD, H), jnp.float32),
        pltpu.SemaphoreType.DMA,
    ],
)


_HALF = EPT_CHUNKS // 2


def _conv_body(y_hbm, src_hbm, dst_hbm, zeros_hbm, out_hbm,
               src_v, dst_v, rows_v, acc, sem_g):
    c = lax.axis_index("c")
    s = lax.axis_index("s")
    row0 = c * ROWS_PER_SC + s * EPT_CHUNKS
    pltpu.sync_copy(zeros_hbm, acc.at[pl.ds(s * RPT, RPT)])
    plsc.subcore_barrier()

    def _gather(j, slot):
        return pltpu.make_async_copy(y_hbm.at[src_v.at[j]],
                                     rows_v.at[slot], sem_g)

    for h in range(2):
        pltpu.sync_copy(src_hbm.at[pl.ds(row0 + h * _HALF, _HALF)], src_v)
        pltpu.sync_copy(dst_hbm.at[pl.ds(row0 + h * _HALF, _HALF)], dst_v)
        _gather(0, 0).start()

        def pair(k, carry):
            j0 = 2 * k
            _gather(j0, 0).wait()
            _gather(j0 + 1, 1).start()
            pltpu.sync_copy(rows_v.at[0], acc.at[dst_v.at[j0]], add=True)
            _gather(j0 + 1, 1).wait()

            @pl.when(k < _HALF // 2 - 1)
            def _():
                _gather(j0 + 2, 0).start()

            pltpu.sync_copy(rows_v.at[1], acc.at[dst_v.at[j0 + 1]], add=True)
            return carry

        lax.fori_loop(0, _HALF // 2, pair, 0)

    plsc.subcore_barrier()
    pltpu.sync_copy(acc.at[pl.ds(s * RPT, RPT)],
                    out_hbm.at[c, pl.ds(s * RPT, RPT)])


def _make_conv(width):
    return pl.kernel(
        _conv_body,
        out_type=jax.ShapeDtypeStruct((NC, NPAD, width), jnp.float32),
        mesh=_MESH,
        scratch_types=[
            pltpu.VMEM((_HALF, CHUNK), jnp.int32),
            pltpu.VMEM((_HALF, CHUNK), jnp.int32),
            pltpu.VMEM((2, CHUNK, width), jnp.float32),
            pltpu.VMEM_SHARED((NPAD, width), jnp.float32),
            pltpu.SemaphoreType.DMA,
        ],
    )


_conv128 = _make_conv(H)

_TC_R = 1000


def _dinv_of(degp):
    deg = degp[0] + degp[1] + 1.0
    return lax.rsqrt(deg[:, 0:1])


def _leaky(x):
    return jnp.where(x > 0, x, 0.01 * x)


def _dense1_body(nodes_ref, w1_ref, b1_ref, gw_ref, y_ref):
    x = jnp.dot(nodes_ref[...], w1_ref[...],
                preferred_element_type=jnp.float32) + b1_ref[...]
    x = _leaky(x)
    y_ref[...] = jnp.dot(x, gw_ref[...],
                         preferred_element_type=jnp.float32)


def _scale_body(y_ref, degp_ref, out_ref):
    out_ref[...] = y_ref[...] * _dinv_of(degp_ref[...])


def _dense2_body(p_ref, y1_ref, degp_ref, w2_ref, b2_ref, g1b_ref, gw2_ref,
                 y2_ref):
    dinv = _dinv_of(degp_ref[...])
    h = (p_ref[0] + p_ref[1] + y1_ref[...]) * dinv + g1b_ref[...]
    x2 = _leaky(jnp.dot(h, w2_ref[...],
                        preferred_element_type=jnp.float32) + b2_ref[...])
    y2_ref[...] = jnp.dot(x2, gw2_ref[...],
                          preferred_element_type=jnp.float32) * dinv


def _final_body(p_ref, y2_ref, degp_ref, g2b_ref, out_ref):
    dinv = _dinv_of(degp_ref[...])
    o = (p_ref[0] + p_ref[1] + y2_ref[...]) * dinv + g2b_ref[...]
    col = lax.broadcasted_iota(jnp.int32, (_TC_R, CP), 1)
    valid = col < C
    m = jnp.max(jnp.where(valid, o, -1e30), axis=1, keepdims=True)
    e = jnp.where(valid, jnp.exp(o - m), 0.0)
    lse = jnp.log(jnp.sum(e, axis=1, keepdims=True))
    out_ref[...] = (o - m - lse)[:, :C]


def _row_spec(width):
    return pl.BlockSpec((_TC_R, width), lambda i: (i, 0))


def _pair_spec(width):
    return pl.BlockSpec((2, _TC_R, width), lambda i: (0, i, 0))


def _full_spec(shape):
    return pl.BlockSpec(shape, lambda i: tuple(0 for _ in shape))


_GRID = (N // _TC_R,)

_dense1_call = pl.pallas_call(
    _dense1_body,
    grid=_GRID,
    in_specs=[_row_spec(D), _full_spec((D, D)), _full_spec((1, D)),
              _full_spec((D, H))],
    out_specs=_row_spec(H),
    out_shape=jax.ShapeDtypeStruct((N, H), jnp.float32),
)

_scale_call = pl.pallas_call(
    _scale_body,
    grid=_GRID,
    in_specs=[_row_spec(H), _pair_spec(LANES)],
    out_specs=_row_spec(H),
    out_shape=jax.ShapeDtypeStruct((N, H), jnp.float32),
)

_dense2_call = pl.pallas_call(
    _dense2_body,
    grid=_GRID,
    in_specs=[_pair_spec(H), _row_spec(H), _pair_spec(LANES),
              _full_spec((H, H)), _full_spec((1, H)), _full_spec((1, H)),
              _full_spec((H, CP))],
    out_specs=_row_spec(CP),
    out_shape=jax.ShapeDtypeStruct((N, CP), jnp.float32),
)

_final_call = pl.pallas_call(
    _final_body,
    grid=_GRID,
    in_specs=[_pair_spec(CP), _row_spec(CP), _pair_spec(LANES),
              _full_spec((1, CP))],
    out_specs=_row_spec(C),
    out_shape=jax.ShapeDtypeStruct((N, C), jnp.float32),
)


def kernel(nodes, edges, lin1_w, lin1_b, bn1_g, bn1_b, bn1_m, bn1_v,
           gcn1_w, gcn1_b, lin2_w, lin2_b, bn2_g, bn2_b, bn2_m, bn2_v,
           gcn2_w, gcn2_b):
    s1 = bn1_g * lax.rsqrt(bn1_v + EPS)
    w1f = lin1_w * s1[None, :]
    b1f = (lin1_b * s1 + (bn1_b - bn1_m * s1))[None, :]
    s2 = bn2_g * lax.rsqrt(bn2_v + EPS)
    w2f = lin2_w * s2[None, :]
    b2f = (lin2_b * s2 + (bn2_b - bn2_m * s2))[None, :]
    gw2p = jnp.pad(gcn2_w, ((0, 0), (0, CP - C)))
    g2bp = jnp.pad(gcn2_b, (0, CP - C))[None, :]

    half_pad = (EPAD - E_RAW) // 2
    half_e = E_RAW // 2
    pad_src = (jnp.arange(half_pad, dtype=jnp.int32) * 7) % N
    pad_dst = N + (jnp.arange(half_pad, dtype=jnp.int32) % (NPAD - N))
    src_flat = jnp.concatenate(
        [edges[:half_e, 0], pad_src, edges[half_e:, 0], pad_src])
    dst_flat = jnp.concatenate(
        [edges[:half_e, 1], pad_dst, edges[half_e:, 1], pad_dst])
    src2d = src_flat.reshape(ROWS_TOTAL, CHUNK)
    dst2d = dst_flat.reshape(ROWS_TOTAL, CHUNK)

    ones128 = jnp.ones((CHUNK, H), jnp.float32)
    z128 = jnp.zeros((RPT, H), jnp.float32)

    degp = _deg_call(dst2d, ones128, z128)
    degp_n = degp[:, :N, :LANES]
    y1u = _dense1_call(nodes, w1f, b1f, gcn1_w)
    y1 = _scale_call(y1u, degp_n)
    p1 = _conv128(y1, src2d, dst2d, z128)
    y2 = _dense2_call(p1[:, :N], y1, degp_n, w2f, b2f,
                      gcn1_b[None, :], gw2p)
    p2 = _conv128(y2, src2d, dst2d, z128)
    return _final_call(p2[:, :N], y2, degp_n, g2bp)


# speedup vs baseline: 23.2577x; 1.0346x over previous
import functools

import jax
import jax.numpy as jnp
from jax import lax
from jax.experimental import pallas as pl
from jax.experimental.pallas import tpu as pltpu
from jax.experimental.pallas import tpu_sc as plsc

N = 10000
D = 128
H = 128
C = 40
CP = 128
EPS = 1e-5

NC, NS, LANES = 2, 16, 16
NPAD = 10240
RPT = NPAD // NS
E_RAW = 320000
CHUNK = 128
EPT_CHUNKS = 80
EPT = CHUNK * EPT_CHUNKS
EPAD = EPT * NC * NS
ROWS_TOTAL = EPAD // CHUNK
ROWS_PER_SC = ROWS_TOTAL // NC

_MESH = plsc.VectorSubcoreMesh(
    core_axis_name="c", subcore_axis_name="s", num_cores=NC, num_subcores=NS)


def _deg_body(dst_hbm, ones_hbm, zeros_hbm, out_hbm, dst_v, ones_v, acc, sem):
    c = lax.axis_index("c")
    s = lax.axis_index("s")
    row0 = c * ROWS_PER_SC + s * EPT_CHUNKS
    pltpu.sync_copy(dst_hbm.at[pl.ds(row0, EPT_CHUNKS)], dst_v)
    pltpu.sync_copy(ones_hbm, ones_v)
    pltpu.sync_copy(zeros_hbm, acc.at[pl.ds(s * RPT, RPT)])
    plsc.subcore_barrier()

    def body(j, carry):
        pltpu.sync_copy(ones_v, acc.at[dst_v.at[j]], add=True)
        return carry

    lax.fori_loop(0, EPT_CHUNKS, body, 0)
    plsc.subcore_barrier()
    pltpu.sync_copy(acc.at[pl.ds(s * RPT, RPT)],
                    out_hbm.at[c, pl.ds(s * RPT, RPT)])


_deg_call = pl.kernel(
    _deg_body,
    out_type=jax.ShapeDtypeStruct((NC, NPAD, H), jnp.float32),
    mesh=_MESH,
    scratch_types=[
        pltpu.VMEM((EPT_CHUNKS, CHUNK), jnp.int32),
        pltpu.VMEM((CHUNK, H), jnp.float32),
        pltpu.VMEM_SHARED((NPAD, H), jnp.float32),
        pltpu.SemaphoreType.DMA,
    ],
)


_HALF = EPT_CHUNKS // 2


def _conv_body(y_hbm, src_hbm, dst_hbm, zeros_hbm, out_hbm,
               src_v, dst_v, rows_v, acc, sem_g):
    c = lax.axis_index("c")
    s = lax.axis_index("s")
    row0 = c * ROWS_PER_SC + s * EPT_CHUNKS
    pltpu.sync_copy(zeros_hbm, acc.at[pl.ds(s * RPT, RPT)])
    plsc.subcore_barrier()

    def _gather(j, slot):
        return pltpu.make_async_copy(y_hbm.at[src_v.at[j]],
                                     rows_v.at[slot], sem_g)

    for h in range(2):
        pltpu.sync_copy(src_hbm.at[pl.ds(row0 + h * _HALF, _HALF)], src_v)
        pltpu.sync_copy(dst_hbm.at[pl.ds(row0 + h * _HALF, _HALF)], dst_v)
        _gather(0, 0).start()

        def pair(k, carry):
            j0 = 2 * k
            _gather(j0, 0).wait()
            _gather(j0 + 1, 1).start()
            pltpu.sync_copy(rows_v.at[0], acc.at[dst_v.at[j0]], add=True)
            _gather(j0 + 1, 1).wait()

            @pl.when(k < _HALF // 2 - 1)
            def _():
                _gather(j0 + 2, 0).start()

            pltpu.sync_copy(rows_v.at[1], acc.at[dst_v.at[j0 + 1]], add=True)
            return carry

        lax.fori_loop(0, _HALF // 2, pair, 0)

    plsc.subcore_barrier()
    pltpu.sync_copy(acc.at[pl.ds(s * RPT, RPT)],
                    out_hbm.at[c, pl.ds(s * RPT, RPT)])


def _make_conv(width):
    return pl.kernel(
        _conv_body,
        out_type=jax.ShapeDtypeStruct((NC, NPAD, width), jnp.float32),
        mesh=_MESH,
        scratch_types=[
            pltpu.VMEM((_HALF, CHUNK), jnp.int32),
            pltpu.VMEM((_HALF, CHUNK), jnp.int32),
            pltpu.VMEM((2, CHUNK, width), jnp.float32),
            pltpu.VMEM_SHARED((NPAD, width), jnp.float32),
            pltpu.SemaphoreType.DMA,
        ],
    )


_conv128 = _make_conv(H)

_TC_R = 1000


def _dinv_of(degp):
    deg = degp[0] + degp[1] + 1.0
    return lax.rsqrt(deg[:, 0:1])


def _leaky(x):
    return jnp.where(x > 0, x, 0.01 * x)


def _dense1_body(nodes_ref, w1_ref, b1_ref, gw_ref, degp_ref, y_ref):
    x = jnp.dot(nodes_ref[...], w1_ref[...],
                preferred_element_type=jnp.float32) + b1_ref[...]
    x = _leaky(x)
    dinv = _dinv_of(degp_ref[...])
    y_ref[...] = jnp.dot(x, gw_ref[...],
                         preferred_element_type=jnp.float32) * dinv


def _dense2_body(p_ref, y1_ref, degp_ref, w2_ref, b2_ref, g1b_ref, gw2_ref,
                 y2_ref):
    dinv = _dinv_of(degp_ref[...])
    h = (p_ref[0] + p_ref[1] + y1_ref[...]) * dinv + g1b_ref[...]
    x2 = _leaky(jnp.dot(h, w2_ref[...],
                        preferred_element_type=jnp.float32) + b2_ref[...])
    y2_ref[...] = jnp.dot(x2, gw2_ref[...],
                          preferred_element_type=jnp.float32) * dinv


def _final_body(p_ref, y2_ref, degp_ref, g2b_ref, out_ref):
    dinv = _dinv_of(degp_ref[...])
    o = (p_ref[0] + p_ref[1] + y2_ref[...]) * dinv + g2b_ref[...]
    col = lax.broadcasted_iota(jnp.int32, (_TC_R, CP), 1)
    valid = col < C
    m = jnp.max(jnp.where(valid, o, -1e30), axis=1, keepdims=True)
    e = jnp.where(valid, jnp.exp(o - m), 0.0)
    lse = jnp.log(jnp.sum(e, axis=1, keepdims=True))
    out_ref[...] = (o - m - lse)[:, :C]


def _row_spec(width):
    return pl.BlockSpec((_TC_R, width), lambda i: (i, 0))


def _pair_spec(width):
    return pl.BlockSpec((2, _TC_R, width), lambda i: (0, i, 0))


def _full_spec(shape):
    return pl.BlockSpec(shape, lambda i: tuple(0 for _ in shape))


_GRID = (N // _TC_R,)

_dense1_call = pl.pallas_call(
    _dense1_body,
    grid=_GRID,
    in_specs=[_row_spec(D), _full_spec((D, D)), _full_spec((1, D)),
              _full_spec((D, H)), _pair_spec(LANES)],
    out_specs=_row_spec(H),
    out_shape=jax.ShapeDtypeStruct((N, H), jnp.float32),
)

_dense2_call = pl.pallas_call(
    _dense2_body,
    grid=_GRID,
    in_specs=[_pair_spec(H), _row_spec(H), _pair_spec(LANES),
              _full_spec((H, H)), _full_spec((1, H)), _full_spec((1, H)),
              _full_spec((H, CP))],
    out_specs=_row_spec(CP),
    out_shape=jax.ShapeDtypeStruct((N, CP), jnp.float32),
)

_final_call = pl.pallas_call(
    _final_body,
    grid=_GRID,
    in_specs=[_pair_spec(CP), _row_spec(CP), _pair_spec(LANES),
              _full_spec((1, CP))],
    out_specs=_row_spec(C),
    out_shape=jax.ShapeDtypeStruct((N, C), jnp.float32),
)


def kernel(nodes, edges, lin1_w, lin1_b, bn1_g, bn1_b, bn1_m, bn1_v,
           gcn1_w, gcn1_b, lin2_w, lin2_b, bn2_g, bn2_b, bn2_m, bn2_v,
           gcn2_w, gcn2_b):
    s1 = bn1_g * lax.rsqrt(bn1_v + EPS)
    w1f = lin1_w * s1[None, :]
    b1f = (lin1_b * s1 + (bn1_b - bn1_m * s1))[None, :]
    s2 = bn2_g * lax.rsqrt(bn2_v + EPS)
    w2f = lin2_w * s2[None, :]
    b2f = (lin2_b * s2 + (bn2_b - bn2_m * s2))[None, :]
    gw2p = jnp.pad(gcn2_w, ((0, 0), (0, CP - C)))
    g2bp = jnp.pad(gcn2_b, (0, CP - C))[None, :]

    half_pad = (EPAD - E_RAW) // 2
    half_e = E_RAW // 2
    pad_src = (jnp.arange(half_pad, dtype=jnp.int32) * 7) % N
    pad_dst = N + (jnp.arange(half_pad, dtype=jnp.int32) % (NPAD - N))
    src_flat = jnp.concatenate(
        [edges[:half_e, 0], pad_src, edges[half_e:, 0], pad_src])
    dst_flat = jnp.concatenate(
        [edges[:half_e, 1], pad_dst, edges[half_e:, 1], pad_dst])
    src2d = src_flat.reshape(ROWS_TOTAL, CHUNK)
    dst2d = dst_flat.reshape(ROWS_TOTAL, CHUNK)

    ones128 = jnp.ones((CHUNK, H), jnp.float32)
    z128 = jnp.zeros((RPT, H), jnp.float32)

    degp = _deg_call(dst2d, ones128, z128)
    degp_n = degp[:, :N, :LANES]
    y1 = _dense1_call(nodes, w1f, b1f, gcn1_w, degp_n)
    p1 = _conv128(y1, src2d, dst2d, z128)
    y2 = _dense2_call(p1, y1, degp_n, w2f, b2f,
                      gcn1_b[None, :], gw2p)
    p2 = _conv128(y2, src2d, dst2d, z128)
    return _final_call(p2, y2, degp_n, g2bp)


# speedup vs baseline: 23.3967x; 1.0060x over previous
import functools

import jax
import jax.numpy as jnp
from jax import lax
from jax.experimental import pallas as pl
from jax.experimental.pallas import tpu as pltpu
from jax.experimental.pallas import tpu_sc as plsc

N = 10000
D = 128
H = 128
C = 40
CP = 128
EPS = 1e-5

NC, NS, LANES = 2, 16, 16
NPAD = 10240
RPT = NPAD // NS
E_RAW = 320000
CHUNK = 128
EPT_CHUNKS = 80
EPT = CHUNK * EPT_CHUNKS
EPAD = EPT * NC * NS
ROWS_TOTAL = EPAD // CHUNK
ROWS_PER_SC = ROWS_TOTAL // NC

_MESH = plsc.VectorSubcoreMesh(
    core_axis_name="c", subcore_axis_name="s", num_cores=NC, num_subcores=NS)


def _deg_body(dst_hbm, ones_hbm, zeros_hbm, out_hbm, dst_v, ones_v, acc, sem):
    c = lax.axis_index("c")
    s = lax.axis_index("s")
    row0 = c * ROWS_PER_SC + s * EPT_CHUNKS
    pltpu.sync_copy(dst_hbm.at[pl.ds(row0, EPT_CHUNKS)], dst_v)
    pltpu.sync_copy(ones_hbm, ones_v)
    pltpu.sync_copy(zeros_hbm, acc.at[pl.ds(s * RPT, RPT)])
    plsc.subcore_barrier()

    def _scat(j):
        return pltpu.make_async_copy(ones_v, acc.at[dst_v.at[j]], sem)

    def body(g, carry):
        j0 = g * 8
        for b in range(8):
            _scat(j0 + b).start(add=True)
        for b in range(8):
            _scat(j0 + b).wait()
        return carry

    lax.fori_loop(0, EPT_CHUNKS // 8, body, 0)
    plsc.subcore_barrier()
    pltpu.sync_copy(acc.at[pl.ds(s * RPT, RPT)],
                    out_hbm.at[c, pl.ds(s * RPT, RPT)])


_deg_call = pl.kernel(
    _deg_body,
    out_type=jax.ShapeDtypeStruct((NC, NPAD, H), jnp.float32),
    mesh=_MESH,
    scratch_types=[
        pltpu.VMEM((EPT_CHUNKS, CHUNK), jnp.int32),
        pltpu.VMEM((CHUNK, H), jnp.float32),
        pltpu.VMEM_SHARED((NPAD, H), jnp.float32),
        pltpu.SemaphoreType.DMA,
    ],
)


_HALF = EPT_CHUNKS // 2


def _conv_body(y_hbm, src_hbm, dst_hbm, zeros_hbm, out_hbm,
               src_v, dst_v, rows_v, acc, sem_g):
    c = lax.axis_index("c")
    s = lax.axis_index("s")
    row0 = c * ROWS_PER_SC + s * EPT_CHUNKS
    pltpu.sync_copy(zeros_hbm, acc.at[pl.ds(s * RPT, RPT)])
    plsc.subcore_barrier()

    def _gather(j, slot):
        return pltpu.make_async_copy(y_hbm.at[src_v.at[j]],
                                     rows_v.at[slot], sem_g)

    for h in range(2):
        pltpu.sync_copy(src_hbm.at[pl.ds(row0 + h * _HALF, _HALF)], src_v)
        pltpu.sync_copy(dst_hbm.at[pl.ds(row0 + h * _HALF, _HALF)], dst_v)
        _gather(0, 0).start()

        def pair(k, carry):
            j0 = 2 * k
            _gather(j0, 0).wait()
            _gather(j0 + 1, 1).start()
            pltpu.sync_copy(rows_v.at[0], acc.at[dst_v.at[j0]], add=True)
            _gather(j0 + 1, 1).wait()

            @pl.when(k < _HALF // 2 - 1)
            def _():
                _gather(j0 + 2, 0).start()

            pltpu.sync_copy(rows_v.at[1], acc.at[dst_v.at[j0 + 1]], add=True)
            return carry

        lax.fori_loop(0, _HALF // 2, pair, 0)

    plsc.subcore_barrier()
    pltpu.sync_copy(acc.at[pl.ds(s * RPT, RPT)],
                    out_hbm.at[c, pl.ds(s * RPT, RPT)])


def _make_conv(width):
    return pl.kernel(
        _conv_body,
        out_type=jax.ShapeDtypeStruct((NC, NPAD, width), jnp.float32),
        mesh=_MESH,
        scratch_types=[
            pltpu.VMEM((_HALF, CHUNK), jnp.int32),
            pltpu.VMEM((_HALF, CHUNK), jnp.int32),
            pltpu.VMEM((2, CHUNK, width), jnp.float32),
            pltpu.VMEM_SHARED((NPAD, width), jnp.float32),
            pltpu.SemaphoreType.DMA,
        ],
    )


_conv128 = _make_conv(H)

_TC_R = 1000


def _dinv_of(degp):
    deg = degp[0] + degp[1] + 1.0
    return lax.rsqrt(deg[:, 0:1])


def _leaky(x):
    return jnp.where(x > 0, x, 0.01 * x)


def _dense1_body(nodes_ref, w1_ref, b1_ref, gw_ref, degp_ref, y_ref):
    x = jnp.dot(nodes_ref[...], w1_ref[...],
                preferred_element_type=jnp.float32) + b1_ref[...]
    x = _leaky(x)
    dinv = _dinv_of(degp_ref[...])
    y_ref[...] = jnp.dot(x, gw_ref[...],
                         preferred_element_type=jnp.float32) * dinv


def _dense2_body(p_ref, y1_ref, degp_ref, w2_ref, b2_ref, g1b_ref, gw2_ref,
                 y2_ref):
    dinv = _dinv_of(degp_ref[...])
    h = (p_ref[0] + p_ref[1] + y1_ref[...]) * dinv + g1b_ref[...]
    x2 = _leaky(jnp.dot(h, w2_ref[...],
                        preferred_element_type=jnp.float32) + b2_ref[...])
    y2_ref[...] = jnp.dot(x2, gw2_ref[...],
                          preferred_element_type=jnp.float32) * dinv


def _final_body(p_ref, y2_ref, degp_ref, g2b_ref, out_ref):
    dinv = _dinv_of(degp_ref[...])
    o = (p_ref[0] + p_ref[1] + y2_ref[...]) * dinv + g2b_ref[...]
    col = lax.broadcasted_iota(jnp.int32, (_TC_R, CP), 1)
    valid = col < C
    m = jnp.max(jnp.where(valid, o, -1e30), axis=1, keepdims=True)
    e = jnp.where(valid, jnp.exp(o - m), 0.0)
    lse = jnp.log(jnp.sum(e, axis=1, keepdims=True))
    out_ref[...] = (o - m - lse)[:, :C]


def _row_spec(width):
    return pl.BlockSpec((_TC_R, width), lambda i: (i, 0))


def _pair_spec(width):
    return pl.BlockSpec((2, _TC_R, width), lambda i: (0, i, 0))


def _full_spec(shape):
    return pl.BlockSpec(shape, lambda i: tuple(0 for _ in shape))


_GRID = (N // _TC_R,)

_dense1_call = pl.pallas_call(
    _dense1_body,
    grid=_GRID,
    in_specs=[_row_spec(D), _full_spec((D, D)), _full_spec((1, D)),
              _full_spec((D, H)), _pair_spec(LANES)],
    out_specs=_row_spec(H),
    out_shape=jax.ShapeDtypeStruct((N, H), jnp.float32),
)

_dense2_call = pl.pallas_call(
    _dense2_body,
    grid=_GRID,
    in_specs=[_pair_spec(H), _row_spec(H), _pair_spec(LANES),
              _full_spec((H, H)), _full_spec((1, H)), _full_spec((1, H)),
              _full_spec((H, CP))],
    out_specs=_row_spec(CP),
    out_shape=jax.ShapeDtypeStruct((N, CP), jnp.float32),
)

_final_call = pl.pallas_call(
    _final_body,
    grid=_GRID,
    in_specs=[_pair_spec(CP), _row_spec(CP), _pair_spec(LANES),
              _full_spec((1, CP))],
    out_specs=_row_spec(C),
    out_shape=jax.ShapeDtypeStruct((N, C), jnp.float32),
)


def kernel(nodes, edges, lin1_w, lin1_b, bn1_g, bn1_b, bn1_m, bn1_v,
           gcn1_w, gcn1_b, lin2_w, lin2_b, bn2_g, bn2_b, bn2_m, bn2_v,
           gcn2_w, gcn2_b):
    s1 = bn1_g * lax.rsqrt(bn1_v + EPS)
    w1f = lin1_w * s1[None, :]
    b1f = (lin1_b * s1 + (bn1_b - bn1_m * s1))[None, :]
    s2 = bn2_g * lax.rsqrt(bn2_v + EPS)
    w2f = lin2_w * s2[None, :]
    b2f = (lin2_b * s2 + (bn2_b - bn2_m * s2))[None, :]
    gw2p = jnp.pad(gcn2_w, ((0, 0), (0, CP - C)))
    g2bp = jnp.pad(gcn2_b, (0, CP - C))[None, :]

    half_pad = (EPAD - E_RAW) // 2
    half_e = E_RAW // 2
    pad_src = (jnp.arange(half_pad, dtype=jnp.int32) * 7) % N
    pad_dst = N + (jnp.arange(half_pad, dtype=jnp.int32) % (NPAD - N))
    src_flat = jnp.concatenate(
        [edges[:half_e, 0], pad_src, edges[half_e:, 0], pad_src])
    dst_flat = jnp.concatenate(
        [edges[:half_e, 1], pad_dst, edges[half_e:, 1], pad_dst])
    src2d = src_flat.reshape(ROWS_TOTAL, CHUNK)
    dst2d = dst_flat.reshape(ROWS_TOTAL, CHUNK)

    ones128 = jnp.ones((CHUNK, H), jnp.float32)
    z128 = jnp.zeros((RPT, H), jnp.float32)

    degp = _deg_call(dst2d, ones128, z128)
    degp_n = degp[:, :N, :LANES]
    y1 = _dense1_call(nodes, w1f, b1f, gcn1_w, degp_n)
    p1 = _conv128(y1, src2d, dst2d, z128)
    y2 = _dense2_call(p1, y1, degp_n, w2f, b2f,
                      gcn1_b[None, :], gw2p)
    p2 = _conv128(y2, src2d, dst2d, z128)
    return _final_call(p2, y2, degp_n, g2bp)
